# baseline (device time: 18420 ns/iter reference)
import jax
import jax.numpy as jnp
from jax import lax
from jax.experimental import pallas as pl
from jax.experimental.pallas import tpu as pltpu

C = 16
QR = 256
R = QR // C
XD = 6 * R
ZF = (6, 7, 8, 9, 10)
YF = (11, 12, 13, 14, 15)
ORDER = (6, 11, 7, 12, 8, 13, 9, 14, 10, 15, 0, 1, 2, 3, 4, 5)
MESH = pl.DeviceIdType.MESH


def kernel(x):
    m, n = x.shape

    def body(x_ref, out_ref, xrecv, by, bz, sx_s, sx_r, sy_s, sy_r,
             sz_s, sz_r, sfy_s, sfy_r, sfz_s, sfz_r):
        my_x = lax.axis_index("x")
        my_y = lax.axis_index("y")
        my_z = lax.axis_index("z")
        xpeer = (1 - my_x, my_y, my_z)
        ypeer = (my_x, 1 - my_y, my_z)
        zpeer = (my_x, my_y, 1 - my_z)
        q = 2 * my_y + my_z
        qy = 2 * (1 - my_y) + my_z
        qz = 2 * my_y + (1 - my_z)
        qd = 2 * (1 - my_y) + (1 - my_z)

        barrier_sem = pltpu.get_barrier_semaphore()
        pl.semaphore_signal(barrier_sem, inc=1, device_id=xpeer,
                            device_id_type=MESH)
        pl.semaphore_signal(by, inc=1, device_id=ypeer,
                            device_id_type=MESH)
        pl.semaphore_signal(bz, inc=1, device_id=zpeer,
                            device_id_type=MESH)
        pl.semaphore_wait(barrier_sem, 1)

        xd = {}
        for c in ORDER:
            d = pltpu.make_async_remote_copy(
                src_ref=x_ref.at[pl.ds(q * QR + c * R, R)],
                dst_ref=xrecv.at[pl.ds(c * R, R)],
                send_sem=sx_s.at[c], recv_sem=sx_r.at[c],
                device_id=xpeer, device_id_type=MESH)
            d.start()
            xd[c] = d
        xde = pltpu.make_async_remote_copy(
            src_ref=x_ref.at[pl.ds(qd * QR, XD)],
            dst_ref=xrecv.at[pl.ds(QR, XD)],
            send_sem=sx_s.at[C], recv_sem=sx_r.at[C],
            device_id=xpeer, device_id_type=MESH)
        xde.start()

        yd, zd = {}, {}
        first = True
        for c in ORDER:
            xd[c].wait()
            rows = pl.ds(q * QR + c * R, R)
            out_ref[rows, :] = x_ref[rows, :] + xrecv[pl.ds(c * R, R), :]
            if first:
                pl.semaphore_wait(by, 1)
                pl.semaphore_wait(bz, 1)
                first = False
            dy = pltpu.make_async_remote_copy(
                src_ref=out_ref.at[rows], dst_ref=out_ref.at[rows],
                send_sem=sy_s.at[c], recv_sem=sy_r.at[c],
                device_id=ypeer, device_id_type=MESH)
            dy.start()
            yd[c] = dy
            dz = pltpu.make_async_remote_copy(
                src_ref=out_ref.at[rows], dst_ref=out_ref.at[rows],
                send_sem=sz_s.at[c], recv_sem=sz_r.at[c],
                device_id=zpeer, device_id_type=MESH)
            dz.start()
            zd[c] = dz

        fzd, fyd = {}, {}
        for c in ORDER:
            ry = pltpu.make_async_remote_copy(
                src_ref=out_ref.at[pl.ds(qy * QR + c * R, R)],
                dst_ref=out_ref.at[pl.ds(qy * QR + c * R, R)],
                send_sem=sy_s.at[c], recv_sem=sy_r.at[c],
                device_id=ypeer, device_id_type=MESH)
            ry.wait_recv()
            if c in ZF:
                fz = pltpu.make_async_remote_copy(
                    src_ref=out_ref.at[pl.ds(qy * QR + c * R, R)],
                    dst_ref=out_ref.at[pl.ds(qy * QR + c * R, R)],
                    send_sem=sfz_s.at[c], recv_sem=sfz_r.at[c],
                    device_id=zpeer, device_id_type=MESH)
                fz.start()
                fzd[c] = fz
            rz = pltpu.make_async_remote_copy(
                src_ref=out_ref.at[pl.ds(qz * QR + c * R, R)],
                dst_ref=out_ref.at[pl.ds(qz * QR + c * R, R)],
                send_sem=sz_s.at[c], recv_sem=sz_r.at[c],
                device_id=zpeer, device_id_type=MESH)
            rz.wait_recv()
            if c in YF:
                fy = pltpu.make_async_remote_copy(
                    src_ref=out_ref.at[pl.ds(qz * QR + c * R, R)],
                    dst_ref=out_ref.at[pl.ds(qz * QR + c * R, R)],
                    send_sem=sfy_s.at[c], recv_sem=sfy_r.at[c],
                    device_id=ypeer, device_id_type=MESH)
                fy.start()
                fyd[c] = fy

        xde.wait()
        out_ref[pl.ds(qd * QR, XD), :] = (
            x_ref[pl.ds(qd * QR, XD), :] + xrecv[pl.ds(QR, XD), :])

        for c in ZF:
            rfz = pltpu.make_async_remote_copy(
                src_ref=out_ref.at[pl.ds(qd * QR + c * R, R)],
                dst_ref=out_ref.at[pl.ds(qd * QR + c * R, R)],
                send_sem=sfz_s.at[c], recv_sem=sfz_r.at[c],
                device_id=zpeer, device_id_type=MESH)
            rfz.wait_recv()
        for c in YF:
            rfy = pltpu.make_async_remote_copy(
                src_ref=out_ref.at[pl.ds(qd * QR + c * R, R)],
                dst_ref=out_ref.at[pl.ds(qd * QR + c * R, R)],
                send_sem=sfy_s.at[c], recv_sem=sfy_r.at[c],
                device_id=ypeer, device_id_type=MESH)
            rfy.wait_recv()
        for c in ORDER:
            yd[c].wait_send()
            zd[c].wait_send()
        for c in ZF:
            fzd[c].wait_send()
        for c in YF:
            fyd[c].wait_send()

    return pl.pallas_call(
        body,
        out_shape=jax.ShapeDtypeStruct((m, n), jnp.float32),
        in_specs=[pl.BlockSpec(memory_space=pltpu.VMEM)],
        out_specs=pl.BlockSpec(memory_space=pltpu.VMEM),
        scratch_shapes=[
            pltpu.VMEM((QR + XD, n), jnp.float32),
            pltpu.SemaphoreType.REGULAR,
            pltpu.SemaphoreType.REGULAR,
            pltpu.SemaphoreType.DMA((C + 1,)), pltpu.SemaphoreType.DMA((C + 1,)),
            pltpu.SemaphoreType.DMA((C,)), pltpu.SemaphoreType.DMA((C,)),
            pltpu.SemaphoreType.DMA((C,)), pltpu.SemaphoreType.DMA((C,)),
            pltpu.SemaphoreType.DMA((C,)), pltpu.SemaphoreType.DMA((C,)),
            pltpu.SemaphoreType.DMA((C,)), pltpu.SemaphoreType.DMA((C,)),
        ],
        compiler_params=pltpu.CompilerParams(collective_id=0),
    )(x)


# device time: 18225 ns/iter; 1.0107x vs baseline; 1.0107x over previous
import jax
import jax.numpy as jnp
from jax import lax
from jax.experimental import pallas as pl
from jax.experimental.pallas import tpu as pltpu

C = 8
QR = 256
R = QR // C
XD = 2 * R
ZF = (2, 3, 4)
YF = (5, 6, 7)
ORDER = (2, 5, 3, 6, 4, 7, 0, 1)
MESH = pl.DeviceIdType.MESH


def kernel(x):
    m, n = x.shape

    def body(x_ref, out_ref, xrecv, by, bz, sx_s, sx_r, sy_s, sy_r,
             sz_s, sz_r, sfy_s, sfy_r, sfz_s, sfz_r):
        my_x = lax.axis_index("x")
        my_y = lax.axis_index("y")
        my_z = lax.axis_index("z")
        xpeer = (1 - my_x, my_y, my_z)
        ypeer = (my_x, 1 - my_y, my_z)
        zpeer = (my_x, my_y, 1 - my_z)
        q = 2 * my_y + my_z
        qy = 2 * (1 - my_y) + my_z
        qz = 2 * my_y + (1 - my_z)
        qd = 2 * (1 - my_y) + (1 - my_z)

        barrier_sem = pltpu.get_barrier_semaphore()
        pl.semaphore_signal(barrier_sem, inc=1, device_id=xpeer,
                            device_id_type=MESH)
        pl.semaphore_signal(by, inc=1, device_id=ypeer,
                            device_id_type=MESH)
        pl.semaphore_signal(bz, inc=1, device_id=zpeer,
                            device_id_type=MESH)
        pl.semaphore_wait(barrier_sem, 1)

        xd = {}
        for c in ORDER:
            d = pltpu.make_async_remote_copy(
                src_ref=x_ref.at[pl.ds(q * QR + c * R, R)],
                dst_ref=xrecv.at[pl.ds(c * R, R)],
                send_sem=sx_s.at[c], recv_sem=sx_r.at[c],
                device_id=xpeer, device_id_type=MESH)
            d.start()
            xd[c] = d
        xde = pltpu.make_async_remote_copy(
            src_ref=x_ref.at[pl.ds(qd * QR, XD)],
            dst_ref=xrecv.at[pl.ds(QR, XD)],
            send_sem=sx_s.at[C], recv_sem=sx_r.at[C],
            device_id=xpeer, device_id_type=MESH)
        xde.start()

        yd, zd = {}, {}
        first = True
        for c in ORDER:
            xd[c].wait()
            rows = pl.ds(q * QR + c * R, R)
            out_ref[rows, :] = x_ref[rows, :] + xrecv[pl.ds(c * R, R), :]
            if first:
                pl.semaphore_wait(by, 1)
                pl.semaphore_wait(bz, 1)
                first = False
            dy = pltpu.make_async_remote_copy(
                src_ref=out_ref.at[rows], dst_ref=out_ref.at[rows],
                send_sem=sy_s.at[c], recv_sem=sy_r.at[c],
                device_id=ypeer, device_id_type=MESH)
            dy.start()
            yd[c] = dy
            dz = pltpu.make_async_remote_copy(
                src_ref=out_ref.at[rows], dst_ref=out_ref.at[rows],
                send_sem=sz_s.at[c], recv_sem=sz_r.at[c],
                device_id=zpeer, device_id_type=MESH)
            dz.start()
            zd[c] = dz

        fzd, fyd = {}, {}
        for c in ORDER:
            ry = pltpu.make_async_remote_copy(
                src_ref=out_ref.at[pl.ds(qy * QR + c * R, R)],
                dst_ref=out_ref.at[pl.ds(qy * QR + c * R, R)],
                send_sem=sy_s.at[c], recv_sem=sy_r.at[c],
                device_id=ypeer, device_id_type=MESH)
            ry.wait_recv()
            if c in ZF:
                fz = pltpu.make_async_remote_copy(
                    src_ref=out_ref.at[pl.ds(qy * QR + c * R, R)],
                    dst_ref=out_ref.at[pl.ds(qy * QR + c * R, R)],
                    send_sem=sfz_s.at[c], recv_sem=sfz_r.at[c],
                    device_id=zpeer, device_id_type=MESH)
                fz.start()
                fzd[c] = fz
            rz = pltpu.make_async_remote_copy(
                src_ref=out_ref.at[pl.ds(qz * QR + c * R, R)],
                dst_ref=out_ref.at[pl.ds(qz * QR + c * R, R)],
                send_sem=sz_s.at[c], recv_sem=sz_r.at[c],
                device_id=zpeer, device_id_type=MESH)
            rz.wait_recv()
            if c in YF:
                fy = pltpu.make_async_remote_copy(
                    src_ref=out_ref.at[pl.ds(qz * QR + c * R, R)],
                    dst_ref=out_ref.at[pl.ds(qz * QR + c * R, R)],
                    send_sem=sfy_s.at[c], recv_sem=sfy_r.at[c],
                    device_id=ypeer, device_id_type=MESH)
                fy.start()
                fyd[c] = fy

        xde.wait()
        out_ref[pl.ds(qd * QR, XD), :] = (
            x_ref[pl.ds(qd * QR, XD), :] + xrecv[pl.ds(QR, XD), :])

        for c in ZF:
            rfz = pltpu.make_async_remote_copy(
                src_ref=out_ref.at[pl.ds(qd * QR + c * R, R)],
                dst_ref=out_ref.at[pl.ds(qd * QR + c * R, R)],
                send_sem=sfz_s.at[c], recv_sem=sfz_r.at[c],
                device_id=zpeer, device_id_type=MESH)
            rfz.wait_recv()
        for c in YF:
            rfy = pltpu.make_async_remote_copy(
                src_ref=out_ref.at[pl.ds(qd * QR + c * R, R)],
                dst_ref=out_ref.at[pl.ds(qd * QR + c * R, R)],
                send_sem=sfy_s.at[c], recv_sem=sfy_r.at[c],
                device_id=ypeer, device_id_type=MESH)
            rfy.wait_recv()
        for c in ORDER:
            yd[c].wait_send()
            zd[c].wait_send()
        for c in ZF:
            fzd[c].wait_send()
        for c in YF:
            fyd[c].wait_send()

    return pl.pallas_call(
        body,
        out_shape=jax.ShapeDtypeStruct((m, n), jnp.float32),
        in_specs=[pl.BlockSpec(memory_space=pltpu.VMEM)],
        out_specs=pl.BlockSpec(memory_space=pltpu.VMEM),
        scratch_shapes=[
            pltpu.VMEM((QR + XD, n), jnp.float32),
            pltpu.SemaphoreType.REGULAR,
            pltpu.SemaphoreType.REGULAR,
            pltpu.SemaphoreType.DMA((C + 1,)), pltpu.SemaphoreType.DMA((C + 1,)),
            pltpu.SemaphoreType.DMA((C,)), pltpu.SemaphoreType.DMA((C,)),
            pltpu.SemaphoreType.DMA((C,)), pltpu.SemaphoreType.DMA((C,)),
            pltpu.SemaphoreType.DMA((C,)), pltpu.SemaphoreType.DMA((C,)),
            pltpu.SemaphoreType.DMA((C,)), pltpu.SemaphoreType.DMA((C,)),
        ],
        compiler_params=pltpu.CompilerParams(collective_id=0),
    )(x)
